# baseline (speedup 1.0000x reference)
import math

import jax
import jax.numpy as jnp
from jax.experimental import pallas as pl
from jax.experimental.pallas import tpu as pltpu

N_HEADS = 16
D_MODEL = 1024
DK = D_MODEL // N_HEADS
DV = DK + 8


def _proj_kernel(x_q, x_k, x_v, wq, wk, wv, bq, bk, bv, oq, ok, ov):
    dn = (((2,), (1,)), ((), ()))
    scale = math.log2(math.e) / math.sqrt(DK)
    xq = x_q[:].astype(jnp.bfloat16)
    xk = x_k[:].astype(jnp.bfloat16)
    xv = x_v[:].astype(jnp.bfloat16)
    oq[:] = ((jax.lax.dot_general(wq[:], xq, dn,
                                  preferred_element_type=jnp.float32)
              + bq[:]) * scale).astype(jnp.bfloat16)
    kh = (jax.lax.dot_general(wk[:], xk, dn,
                              preferred_element_type=jnp.float32)
          + bk[:]).astype(jnp.bfloat16)
    vh = (jax.lax.dot_general(wv[:], xv, dn,
                              preferred_element_type=jnp.float32)
          + bv[:]).astype(jnp.bfloat16)
    pad_shape = (N_HEADS, DV - DK, vh.shape[2])
    row = jax.lax.broadcasted_iota(jnp.int32, pad_shape, 1)
    ones_row = jnp.where(row == 0, 1.0, 0.0).astype(jnp.bfloat16)
    ok[:] = jnp.concatenate([kh, ones_row], axis=1)
    ov[:] = jnp.concatenate([vh, ones_row], axis=1)


def _attn_kernel(q_ref, k_ref, v_ref, o_ref):
    q = q_ref[0]
    k = k_ref[0]
    qf = q.astype(jnp.float32)
    kf = k[:DK, :].astype(jnp.float32)
    kn = jnp.sqrt(jnp.max(jnp.sum(kf * kf, axis=0)))
    qn = jnp.sqrt(jnp.sum(qf * qf, axis=0, keepdims=True))
    shift = (-qn * kn).astype(jnp.bfloat16)
    pad = jnp.concatenate(
        [shift, jnp.zeros((DV - DK - 1, shift.shape[1]), jnp.bfloat16)], axis=0)
    q_aug = jnp.concatenate([q, pad], axis=0)
    st = jax.lax.dot_general(k, q_aug, (((0,), (0,)), ((), ())),
                             preferred_element_type=jnp.float32)
    p = jnp.exp2(st).astype(jnp.bfloat16)
    o_ref[0] = jax.lax.dot_general(
        v_ref[0], p, (((1,), (0,)), ((), ())),
        preferred_element_type=jnp.float32).astype(jnp.bfloat16)


def _oproj_kernel(x_ref, wo_ref, bo_ref, o_ref):
    bm = x_ref.shape[2]
    x = x_ref[:].astype(jnp.float32)
    ctx = x[:, :DK, :] / x[:, DK:DK + 1, :]
    ctxn = ctx.astype(jnp.bfloat16).reshape(D_MODEL, bm)
    o_ref[:] = jax.lax.dot_general(ctxn, wo_ref[:], (((0,), (1,)), ((), ())),
                                   preferred_element_type=jnp.float32) + bo_ref[:]


def kernel(Q, K, V, Wq, bq, Wk, bk, Wv, bv, Wo, bo):
    B, L, D = Q.shape
    H, dk = N_HEADS, DK
    bf = jnp.bfloat16
    x_q = Q.reshape(L, D)
    x_k = K.reshape(L, D)
    x_v = V.reshape(L, D)
    wq3 = Wq.reshape(H, dk, D).astype(bf)
    wk3 = Wk.reshape(H, dk, D).astype(bf)
    wv3 = Wv.reshape(H, dk, D).astype(bf)
    bq3 = bq.reshape(H, dk, 1)
    bk3 = bk.reshape(H, dk, 1)
    bv3 = bv.reshape(H, dk, 1)
    bo2 = bo.reshape(1, D)

    BM = 512
    n_rb = L // BM

    w3_spec = pl.BlockSpec((H, dk, D), lambda i: (0, 0, 0))
    b3_spec = pl.BlockSpec((H, dk, 1), lambda i: (0, 0, 0))
    row_spec = pl.BlockSpec((BM, D), lambda i: (i, 0))
    headsT_spec = pl.BlockSpec((H, dk, BM), lambda i: (0, 0, i))
    headsTv_spec = pl.BlockSpec((H, DV, BM), lambda i: (0, 0, i))

    qp, kp, vp = pl.pallas_call(
        _proj_kernel,
        grid=(n_rb,),
        compiler_params=pltpu.CompilerParams(
            dimension_semantics=("parallel",)),
        in_specs=[row_spec, row_spec, row_spec,
                  w3_spec, w3_spec, w3_spec,
                  b3_spec, b3_spec, b3_spec],
        out_specs=[headsT_spec, headsTv_spec, headsTv_spec],
        out_shape=[jax.ShapeDtypeStruct((H, dk, L), bf),
                   jax.ShapeDtypeStruct((H, DV, L), bf),
                   jax.ShapeDtypeStruct((H, DV, L), bf)],
    )(x_q, x_k, x_v, wq3, wk3, wv3, bq3, bk3, bv3)

    BQ = 2048
    n_qb = L // BQ
    ctx = pl.pallas_call(
        _attn_kernel,
        grid=(H, n_qb),
        compiler_params=pltpu.CompilerParams(
            dimension_semantics=("parallel", "parallel")),
        in_specs=[
            pl.BlockSpec((1, dk, BQ), lambda h, qb: (h, 0, qb)),
            pl.BlockSpec((1, DV, L), lambda h, qb: (h, 0, 0)),
            pl.BlockSpec((1, DV, L), lambda h, qb: (h, 0, 0)),
        ],
        out_specs=pl.BlockSpec((1, DV, BQ), lambda h, qb: (h, 0, qb)),
        out_shape=jax.ShapeDtypeStruct((H, DV, L), bf),
    )(qp, kp, vp)

    wo_spec = pl.BlockSpec((D, D), lambda i: (0, 0))
    b_spec = pl.BlockSpec((1, D), lambda i: (0, 0))
    out = pl.pallas_call(
        _oproj_kernel,
        grid=(n_rb,),
        compiler_params=pltpu.CompilerParams(
            dimension_semantics=("parallel",)),
        in_specs=[headsTv_spec, wo_spec, b_spec],
        out_specs=row_spec,
        out_shape=jax.ShapeDtypeStruct((L, D), jnp.float32),
    )(ctx, Wo.astype(bf), bo2)

    return out.reshape(B, L, D)


# speedup vs baseline: 1.0509x; 1.0509x over previous
import math

import jax
import jax.numpy as jnp
from jax.experimental import pallas as pl
from jax.experimental.pallas import tpu as pltpu

N_HEADS = 16
D_MODEL = 1024
DK = D_MODEL // N_HEADS
DV = DK + 8


def _proj_kernel(x_q, x_k, x_v, wq, wk, wv, bq, bk, bv, oq, ok, ov):
    dn = (((2,), (1,)), ((), ()))
    scale = math.log2(math.e) / math.sqrt(DK)
    xq = x_q[:].astype(jnp.bfloat16)
    xk = x_k[:].astype(jnp.bfloat16)
    xv = x_v[:].astype(jnp.bfloat16)
    oq[:] = ((jax.lax.dot_general(wq[:], xq, dn,
                                  preferred_element_type=jnp.float32)
              + bq[:]) * scale).astype(jnp.bfloat16)
    kh = (jax.lax.dot_general(wk[:], xk, dn,
                              preferred_element_type=jnp.float32)
          + bk[:]).astype(jnp.bfloat16)
    vh = (jax.lax.dot_general(wv[:], xv, dn,
                              preferred_element_type=jnp.float32)
          + bv[:]).astype(jnp.bfloat16)
    pad_shape = (N_HEADS, DV - DK, vh.shape[2])
    row = jax.lax.broadcasted_iota(jnp.int32, pad_shape, 1)
    ones_row = jnp.where(row == 0, 1.0, 0.0).astype(jnp.bfloat16)
    ok[:] = jnp.concatenate([kh, ones_row], axis=1)
    ov[:] = jnp.concatenate([vh, ones_row], axis=1)


def _attn_kernel(q_ref, k_ref, v_ref, o_ref):
    q = q_ref[0]
    k = k_ref[0]
    qf = q.astype(jnp.float32)
    kf = k[:DK, :].astype(jnp.float32)
    kn = jnp.sqrt(jnp.max(jnp.sum(kf * kf, axis=0)))
    qn = jnp.sqrt(jnp.sum(qf * qf, axis=0, keepdims=True))
    shift = (-qn * kn).astype(jnp.bfloat16)
    pad = jnp.concatenate(
        [shift, jnp.zeros((DV - DK - 1, shift.shape[1]), jnp.bfloat16)], axis=0)
    q_aug = jnp.concatenate([q, pad], axis=0)
    st = jax.lax.dot_general(k, q_aug, (((0,), (0,)), ((), ())),
                             preferred_element_type=jnp.float32)
    p = jnp.exp2(st).astype(jnp.bfloat16)
    o_ref[0] = jax.lax.dot_general(
        v_ref[0], p, (((1,), (0,)), ((), ())),
        preferred_element_type=jnp.float32).astype(jnp.bfloat16)


def _oproj_kernel(x_ref, wo_ref, bo_ref, o_ref):
    bm = x_ref.shape[2]
    x = x_ref[:].astype(jnp.float32)
    ctx = x[:, :DK, :] / x[:, DK:DK + 1, :]
    ctxn = ctx.astype(jnp.bfloat16).reshape(D_MODEL, bm)
    o_ref[:] = jax.lax.dot_general(ctxn, wo_ref[:], (((0,), (1,)), ((), ())),
                                   preferred_element_type=jnp.float32) + bo_ref[:]


def kernel(Q, K, V, Wq, bq, Wk, bk, Wv, bv, Wo, bo):
    B, L, D = Q.shape
    H, dk = N_HEADS, DK
    bf = jnp.bfloat16
    x_q = Q.reshape(L, D)
    x_k = K.reshape(L, D)
    x_v = V.reshape(L, D)
    wq3 = Wq.reshape(H, dk, D).astype(bf)
    wk3 = Wk.reshape(H, dk, D).astype(bf)
    wv3 = Wv.reshape(H, dk, D).astype(bf)
    bq3 = bq.reshape(H, dk, 1)
    bk3 = bk.reshape(H, dk, 1)
    bv3 = bv.reshape(H, dk, 1)
    bo2 = bo.reshape(1, D)

    BM = 512
    n_rb = L // BM

    w3_spec = pl.BlockSpec((H, dk, D), lambda i: (0, 0, 0))
    b3_spec = pl.BlockSpec((H, dk, 1), lambda i: (0, 0, 0))
    row_spec = pl.BlockSpec((BM, D), lambda i: (i, 0))
    headsT_spec = pl.BlockSpec((H, dk, BM), lambda i: (0, 0, i))
    headsTv_spec = pl.BlockSpec((H, DV, BM), lambda i: (0, 0, i))

    qp, kp, vp = pl.pallas_call(
        _proj_kernel,
        grid=(n_rb,),
        compiler_params=pltpu.CompilerParams(
            dimension_semantics=("parallel",),
            allow_input_fusion=[True] * 9),
        in_specs=[row_spec, row_spec, row_spec,
                  w3_spec, w3_spec, w3_spec,
                  b3_spec, b3_spec, b3_spec],
        out_specs=[headsT_spec, headsTv_spec, headsTv_spec],
        out_shape=[jax.ShapeDtypeStruct((H, dk, L), bf),
                   jax.ShapeDtypeStruct((H, DV, L), bf),
                   jax.ShapeDtypeStruct((H, DV, L), bf)],
    )(x_q, x_k, x_v, wq3, wk3, wv3, bq3, bk3, bv3)

    BQ = 2048
    n_qb = L // BQ
    ctx = pl.pallas_call(
        _attn_kernel,
        grid=(H, n_qb),
        compiler_params=pltpu.CompilerParams(
            dimension_semantics=("parallel", "parallel")),
        in_specs=[
            pl.BlockSpec((1, dk, BQ), lambda h, qb: (h, 0, qb)),
            pl.BlockSpec((1, DV, L), lambda h, qb: (h, 0, 0)),
            pl.BlockSpec((1, DV, L), lambda h, qb: (h, 0, 0)),
        ],
        out_specs=pl.BlockSpec((1, DV, BQ), lambda h, qb: (h, 0, qb)),
        out_shape=jax.ShapeDtypeStruct((H, DV, L), bf),
    )(qp, kp, vp)

    wo_spec = pl.BlockSpec((D, D), lambda i: (0, 0))
    b_spec = pl.BlockSpec((1, D), lambda i: (0, 0))
    out = pl.pallas_call(
        _oproj_kernel,
        grid=(n_rb,),
        compiler_params=pltpu.CompilerParams(
            dimension_semantics=("parallel",),
            allow_input_fusion=[True] * 3),
        in_specs=[headsTv_spec, wo_spec, b_spec],
        out_specs=row_spec,
        out_shape=jax.ShapeDtypeStruct((L, D), jnp.float32),
    )(ctx, Wo.astype(bf), bo2)

    return out.reshape(B, L, D)
